# baseline (speedup 1.0000x reference)
import jax
import jax.numpy as jnp
from jax.experimental import pallas as pl
from jax.experimental.pallas import tpu as pltpu

DIM = 2048
NH = 16
QLORA = 1536
KVLORA = 512
NOPE = 128
ROPE = 64
VDIM = 128
QK = NOPE + ROPE
S = 2048
EPS = 1e-6
SCALE = QK ** (-0.5)

f32 = jnp.float32
bf16 = jnp.bfloat16

NT = (((1,), (1,)), ((), ()))

BSA = 1024
BSB = 512
BSC = 512
BQ = 512
BK = 512
BSO = 1024


def _rope(x, c, s1, s2):
    w = x.shape[-1]
    return (x * c + pltpu.roll(x, w - 1, 1) * s1 + pltpu.roll(x, 1, 1) * s2)


def _qa_kernel(x_ref, wqa_ref, gqa_ref, out_ref, wqa_s):
    @pl.when(pl.program_id(0) == 0)
    def _cast():
        wqa_s[...] = wqa_ref[...].astype(bf16)

    xb = x_ref[...].astype(bf16)
    qa = jax.lax.dot_general(xb, wqa_s[...], NT, preferred_element_type=f32)
    var = jnp.mean(qa * qa, axis=-1, keepdims=True)
    out_ref[...] = ((qa * jax.lax.rsqrt(var + EPS)) * gqa_ref[...]).astype(bf16)


def _qb_kernel(qa_ref, wqb_ref, out_ref, wqb_s):
    @pl.when(pl.program_id(0) == 0)
    def _cast():
        wqb_s[...] = wqb_ref[...].astype(bf16)

    out_ref[...] = jax.lax.dot_general(
        qa_ref[...], wqb_s[...], NT, preferred_element_type=f32).astype(bf16)


def _kv_kernel(x_ref, wkva_ref, wkvb_ref, gkv_ref, c_ref, s1_ref, s2_ref,
               kn_ref, kp_ref, v_ref, wkva_s, wkvb_s):
    @pl.when(pl.program_id(0) == 0)
    def _cast():
        wkva_s[...] = wkva_ref[...].astype(bf16)
        wkvb_s[...] = wkvb_ref[...].astype(bf16)

    xb = x_ref[...].astype(bf16)
    kva = jax.lax.dot_general(xb, wkva_s[...], NT, preferred_element_type=f32)
    lat = kva[:, :KVLORA]
    kpe = _rope(kva[:, KVLORA:], c_ref[...], s1_ref[...], s2_ref[...])
    kp_ref[...] = kpe.astype(bf16)
    var = jnp.mean(lat * lat, axis=-1, keepdims=True)
    latb = ((lat * jax.lax.rsqrt(var + EPS)) * gkv_ref[...]).astype(bf16)
    kvb = jax.lax.dot_general(latb, wkvb_s[...], NT, preferred_element_type=f32)
    for h in range(NH):
        kn_ref[:, h * NOPE:(h + 1) * NOPE] = (
            kvb[:, h * (NOPE + VDIM):h * (NOPE + VDIM) + NOPE].astype(bf16))
        v_ref[:, h * VDIM:(h + 1) * VDIM] = (
            kvb[:, h * (NOPE + VDIM) + NOPE:(h + 1) * (NOPE + VDIM)]
            .astype(bf16))


def _attn_kernel(q_ref, kn_ref, kp_ref, v_ref, c_ref, s1_ref, s2_ref, o_ref):
    kp = kp_ref[...]
    c = c_ref[...]
    s1 = s1_ref[...]
    s2 = s2_ref[...]
    for hh in range(2):
        qh_raw = q_ref[:, hh * QK:(hh + 1) * QK].astype(f32)
        qh = _rope(qh_raw, c, s1, s2).astype(bf16)
        kh = jnp.concatenate(
            [kn_ref[:, hh * NOPE:(hh + 1) * NOPE], kp], axis=1)
        vh = v_ref[:, hh * VDIM:(hh + 1) * VDIM]
        outs = []
        for i in range(S // BQ):
            q_i = qh[i * BQ:(i + 1) * BQ, :]
            svals = []
            for j in range(i + 1):
                k_j = kh[j * BK:(j + 1) * BK, :]
                sc = jax.lax.dot_general(q_i, k_j, NT,
                                         preferred_element_type=f32) * SCALE
                if j == i:
                    r = jax.lax.broadcasted_iota(jnp.int32, (BQ, BK), 0)
                    cidx = jax.lax.broadcasted_iota(jnp.int32, (BQ, BK), 1)
                    sc = jnp.where(cidx <= r, sc, -1e30)
                svals.append(sc)
            m = svals[0].max(axis=1, keepdims=True)
            for sv in svals[1:]:
                m = jnp.maximum(m, sv.max(axis=1, keepdims=True))
            ps = [jnp.exp(sv - m) for sv in svals]
            l = ps[0].sum(axis=1, keepdims=True)
            for p in ps[1:]:
                l = l + p.sum(axis=1, keepdims=True)
            acc = jnp.dot(ps[0].astype(bf16), vh[0:BK, :],
                          preferred_element_type=f32)
            for j in range(1, i + 1):
                acc = acc + jnp.dot(ps[j].astype(bf16),
                                    vh[j * BK:(j + 1) * BK, :],
                                    preferred_element_type=f32)
            outs.append((acc * (1.0 / l)).astype(bf16))
        o_ref[:, hh * VDIM:(hh + 1) * VDIM] = jnp.concatenate(outs, axis=0)


def _oproj_kernel(o_ref, wo_ref, out_ref, wo_s):
    @pl.when(pl.program_id(0) == 0)
    def _cast():
        wo_s[...] = wo_ref[...].astype(bf16)

    out_ref[...] = jax.lax.dot_general(o_ref[...], wo_s[...], NT,
                                       preferred_element_type=f32)


def kernel(x, freqs_cos, freqs_sin, mask, W_qa, g_qa, W_qb, W_kva, g_kv,
           W_kvb, W_o):
    del mask
    b, s, _ = x.shape
    x2 = x.reshape(s, DIM)
    gqa2 = g_qa.reshape(1, QLORA)
    gkv2 = g_kv.reshape(1, KVLORA)

    c64 = jnp.repeat(freqs_cos, 2, axis=-1)
    s64 = jnp.repeat(freqs_sin, 2, axis=-1)
    even = (jnp.arange(ROPE) % 2 == 0)
    s1_64 = jnp.where(even, -s64, 0.0)
    s2_64 = jnp.where(even, 0.0, s64)
    ones_n = jnp.ones((S, NOPE), f32)
    zero_n = jnp.zeros((S, NOPE), f32)
    c192 = jnp.concatenate([ones_n, c64], axis=1)
    s1_192 = jnp.concatenate([zero_n, s1_64], axis=1)
    s2_192 = jnp.concatenate([zero_n, s2_64], axis=1)

    qa_n = pl.pallas_call(
        _qa_kernel,
        grid=(S // BSA,),
        in_specs=[
            pl.BlockSpec((BSA, DIM), lambda i: (i, 0)),
            pl.BlockSpec((QLORA, DIM), lambda i: (0, 0)),
            pl.BlockSpec((1, QLORA), lambda i: (0, 0)),
        ],
        out_specs=pl.BlockSpec((BSA, QLORA), lambda i: (i, 0)),
        out_shape=jax.ShapeDtypeStruct((S, QLORA), bf16),
        scratch_shapes=[pltpu.VMEM((QLORA, DIM), bf16)],
    )(x2, W_qa, gqa2)

    q_int = pl.pallas_call(
        _qb_kernel,
        grid=(S // BSB,),
        in_specs=[
            pl.BlockSpec((BSB, QLORA), lambda i: (i, 0)),
            pl.BlockSpec((NH * QK, QLORA), lambda i: (0, 0)),
        ],
        out_specs=pl.BlockSpec((BSB, NH * QK), lambda i: (i, 0)),
        out_shape=jax.ShapeDtypeStruct((S, NH * QK), bf16),
        scratch_shapes=[pltpu.VMEM((NH * QK, QLORA), bf16)],
    )(qa_n, W_qb)

    kn, kp, v = pl.pallas_call(
        _kv_kernel,
        grid=(S // BSC,),
        in_specs=[
            pl.BlockSpec((BSC, DIM), lambda i: (i, 0)),
            pl.BlockSpec((KVLORA + ROPE, DIM), lambda i: (0, 0)),
            pl.BlockSpec((NH * (NOPE + VDIM), KVLORA), lambda i: (0, 0)),
            pl.BlockSpec((1, KVLORA), lambda i: (0, 0)),
            pl.BlockSpec((BSC, ROPE), lambda i: (i, 0)),
            pl.BlockSpec((BSC, ROPE), lambda i: (i, 0)),
            pl.BlockSpec((BSC, ROPE), lambda i: (i, 0)),
        ],
        out_specs=[
            pl.BlockSpec((BSC, NH * NOPE), lambda i: (i, 0)),
            pl.BlockSpec((BSC, ROPE), lambda i: (i, 0)),
            pl.BlockSpec((BSC, NH * VDIM), lambda i: (i, 0)),
        ],
        out_shape=[
            jax.ShapeDtypeStruct((S, NH * NOPE), bf16),
            jax.ShapeDtypeStruct((S, ROPE), bf16),
            jax.ShapeDtypeStruct((S, NH * VDIM), bf16),
        ],
        scratch_shapes=[
            pltpu.VMEM((KVLORA + ROPE, DIM), bf16),
            pltpu.VMEM((NH * (NOPE + VDIM), KVLORA), bf16),
        ],
    )(x2, W_kva, W_kvb, gkv2, c64, s1_64, s2_64)

    o = pl.pallas_call(
        _attn_kernel,
        grid=(NH // 2,),
        in_specs=[
            pl.BlockSpec((S, 2 * QK), lambda h: (0, h)),
            pl.BlockSpec((S, 2 * NOPE), lambda h: (0, h)),
            pl.BlockSpec((S, ROPE), lambda h: (0, 0)),
            pl.BlockSpec((S, 2 * VDIM), lambda h: (0, h)),
            pl.BlockSpec((S, QK), lambda h: (0, 0)),
            pl.BlockSpec((S, QK), lambda h: (0, 0)),
            pl.BlockSpec((S, QK), lambda h: (0, 0)),
        ],
        out_specs=pl.BlockSpec((S, 2 * VDIM), lambda h: (0, h)),
        out_shape=jax.ShapeDtypeStruct((S, NH * VDIM), bf16),
    )(q_int, kn, kp, v, c192, s1_192, s2_192)

    out = pl.pallas_call(
        _oproj_kernel,
        grid=(S // BSO,),
        in_specs=[
            pl.BlockSpec((BSO, NH * VDIM), lambda i: (i, 0)),
            pl.BlockSpec((DIM, NH * VDIM), lambda i: (0, 0)),
        ],
        out_specs=pl.BlockSpec((BSO, DIM), lambda i: (i, 0)),
        out_shape=jax.ShapeDtypeStruct((S, DIM), f32),
        scratch_shapes=[pltpu.VMEM((DIM, NH * VDIM), bf16)],
    )(o, W_o)

    return out.reshape(b, s, DIM)


# speedup vs baseline: 1.2436x; 1.2436x over previous
import jax
import jax.numpy as jnp
from jax.experimental import pallas as pl
from jax.experimental.pallas import tpu as pltpu

DIM = 2048
NH = 16
QLORA = 1536
KVLORA = 512
NOPE = 128
ROPE = 64
VDIM = 128
QK = NOPE + ROPE
S = 2048
EPS = 1e-6
SCALE = QK ** (-0.5)

f32 = jnp.float32
bf16 = jnp.bfloat16

NT = (((1,), (1,)), ((), ()))

BSA = 1024
BSB = 512
BSC = 512
BQ = 512
BK = 512
BSO = 1024


def _rope(x, c, s1, s2):
    w = x.shape[-1]
    return (x * c + pltpu.roll(x, w - 1, 1) * s1 + pltpu.roll(x, 1, 1) * s2)


def _qa_kernel(x_ref, wqa_ref, gqa_ref, out_ref, wqa_s):
    @pl.when(pl.program_id(0) == 0)
    def _cast():
        wqa_s[...] = wqa_ref[...].astype(bf16)

    xb = x_ref[...].astype(bf16)
    qa = jax.lax.dot_general(xb, wqa_s[...], NT, preferred_element_type=f32)
    var = jnp.mean(qa * qa, axis=-1, keepdims=True)
    out_ref[...] = ((qa * jax.lax.rsqrt(var + EPS)) * gqa_ref[...]).astype(bf16)


QBN = 768


def _qb_kernel(qa_ref, wqb_ref, c_ref, s1_ref, s2_ref, out_ref):
    wb = wqb_ref[...].astype(bf16)
    q = jax.lax.dot_general(qa_ref[...], wb, NT, preferred_element_type=f32)
    c = jnp.tile(c_ref[...], (1, QBN // QK))
    s1 = jnp.tile(s1_ref[...], (1, QBN // QK))
    s2 = jnp.tile(s2_ref[...], (1, QBN // QK))
    out_ref[...] = _rope(q, c, s1, s2).astype(bf16)


def _kv_kernel(x_ref, wkva_ref, wkvb_ref, gkv_ref, c_ref, s1_ref, s2_ref,
               kn_ref, kp_ref, v_ref, wkva_s, wkvb_s):
    @pl.when(pl.program_id(0) == 0)
    def _cast():
        wkva_s[...] = wkva_ref[...].astype(bf16)
        wkvb_s[...] = wkvb_ref[...].astype(bf16)

    xb = x_ref[...].astype(bf16)
    kva = jax.lax.dot_general(xb, wkva_s[...], NT, preferred_element_type=f32)
    lat = kva[:, :KVLORA]
    kpe = _rope(kva[:, KVLORA:], c_ref[...], s1_ref[...], s2_ref[...])
    kp_ref[...] = kpe.astype(bf16)
    var = jnp.mean(lat * lat, axis=-1, keepdims=True)
    latb = ((lat * jax.lax.rsqrt(var + EPS)) * gkv_ref[...]).astype(bf16)
    kvb = jax.lax.dot_general(latb, wkvb_s[...], NT, preferred_element_type=f32)
    for h in range(NH):
        kn_ref[:, h * NOPE:(h + 1) * NOPE] = (
            kvb[:, h * (NOPE + VDIM):h * (NOPE + VDIM) + NOPE].astype(bf16))
        v_ref[:, h * VDIM:(h + 1) * VDIM] = (
            kvb[:, h * (NOPE + VDIM) + NOPE:(h + 1) * (NOPE + VDIM)]
            .astype(bf16))


def _attn_kernel(q_ref, kn_ref, kp_ref, v_ref, o_ref):
    kp = kp_ref[...]
    for hh in range(2):
        qh = q_ref[:, hh * QK:(hh + 1) * QK]
        kh = jnp.concatenate(
            [kn_ref[:, hh * NOPE:(hh + 1) * NOPE], kp], axis=1)
        vh = v_ref[:, hh * VDIM:(hh + 1) * VDIM]
        outs = []
        for i in range(S // BQ):
            q_i = qh[i * BQ:(i + 1) * BQ, :]
            svals = []
            for j in range(i + 1):
                k_j = kh[j * BK:(j + 1) * BK, :]
                sc = jax.lax.dot_general(q_i, k_j, NT,
                                         preferred_element_type=f32) * SCALE
                if j == i:
                    r = jax.lax.broadcasted_iota(jnp.int32, (BQ, BK), 0)
                    cidx = jax.lax.broadcasted_iota(jnp.int32, (BQ, BK), 1)
                    sc = jnp.where(cidx <= r, sc, -1e30)
                svals.append(sc)
            m = svals[0].max(axis=1, keepdims=True)
            for sv in svals[1:]:
                m = jnp.maximum(m, sv.max(axis=1, keepdims=True))
            ps = [jnp.exp(sv - m) for sv in svals]
            l = ps[0].sum(axis=1, keepdims=True)
            for p in ps[1:]:
                l = l + p.sum(axis=1, keepdims=True)
            acc = jnp.dot(ps[0].astype(bf16), vh[0:BK, :],
                          preferred_element_type=f32)
            for j in range(1, i + 1):
                acc = acc + jnp.dot(ps[j].astype(bf16),
                                    vh[j * BK:(j + 1) * BK, :],
                                    preferred_element_type=f32)
            outs.append((acc * (1.0 / l)).astype(bf16))
        o_ref[:, hh * VDIM:(hh + 1) * VDIM] = jnp.concatenate(outs, axis=0)


def _oproj_kernel(o_ref, wo_ref, out_ref, wo_s):
    @pl.when(pl.program_id(0) == 0)
    def _cast():
        wo_s[...] = wo_ref[...].astype(bf16)

    out_ref[...] = jax.lax.dot_general(o_ref[...], wo_s[...], NT,
                                       preferred_element_type=f32)


def kernel(x, freqs_cos, freqs_sin, mask, W_qa, g_qa, W_qb, W_kva, g_kv,
           W_kvb, W_o):
    del mask
    b, s, _ = x.shape
    x2 = x.reshape(s, DIM)
    gqa2 = g_qa.reshape(1, QLORA)
    gkv2 = g_kv.reshape(1, KVLORA)

    c64 = jnp.repeat(freqs_cos, 2, axis=-1)
    s64 = jnp.repeat(freqs_sin, 2, axis=-1)
    even = (jnp.arange(ROPE) % 2 == 0)
    s1_64 = jnp.where(even, -s64, 0.0)
    s2_64 = jnp.where(even, 0.0, s64)
    ones_n = jnp.ones((S, NOPE), f32)
    zero_n = jnp.zeros((S, NOPE), f32)
    c192 = jnp.concatenate([ones_n, c64], axis=1)
    s1_192 = jnp.concatenate([zero_n, s1_64], axis=1)
    s2_192 = jnp.concatenate([zero_n, s2_64], axis=1)

    qa_n = pl.pallas_call(
        _qa_kernel,
        grid=(S // BSA,),
        in_specs=[
            pl.BlockSpec((BSA, DIM), lambda i: (i, 0)),
            pl.BlockSpec((QLORA, DIM), lambda i: (0, 0)),
            pl.BlockSpec((1, QLORA), lambda i: (0, 0)),
        ],
        out_specs=pl.BlockSpec((BSA, QLORA), lambda i: (i, 0)),
        out_shape=jax.ShapeDtypeStruct((S, QLORA), bf16),
        scratch_shapes=[pltpu.VMEM((QLORA, DIM), bf16)],
    )(x2, W_qa, gqa2)

    q_int = pl.pallas_call(
        _qb_kernel,
        grid=(NH * QK // QBN, S // BSB),
        in_specs=[
            pl.BlockSpec((BSB, QLORA), lambda n, r: (r, 0)),
            pl.BlockSpec((QBN, QLORA), lambda n, r: (n, 0)),
            pl.BlockSpec((BSB, QK), lambda n, r: (r, 0)),
            pl.BlockSpec((BSB, QK), lambda n, r: (r, 0)),
            pl.BlockSpec((BSB, QK), lambda n, r: (r, 0)),
        ],
        out_specs=pl.BlockSpec((BSB, QBN), lambda n, r: (r, n)),
        out_shape=jax.ShapeDtypeStruct((S, NH * QK), bf16),
    )(qa_n, W_qb, c192, s1_192, s2_192)

    kn, kp, v = pl.pallas_call(
        _kv_kernel,
        grid=(S // BSC,),
        in_specs=[
            pl.BlockSpec((BSC, DIM), lambda i: (i, 0)),
            pl.BlockSpec((KVLORA + ROPE, DIM), lambda i: (0, 0)),
            pl.BlockSpec((NH * (NOPE + VDIM), KVLORA), lambda i: (0, 0)),
            pl.BlockSpec((1, KVLORA), lambda i: (0, 0)),
            pl.BlockSpec((BSC, ROPE), lambda i: (i, 0)),
            pl.BlockSpec((BSC, ROPE), lambda i: (i, 0)),
            pl.BlockSpec((BSC, ROPE), lambda i: (i, 0)),
        ],
        out_specs=[
            pl.BlockSpec((BSC, NH * NOPE), lambda i: (i, 0)),
            pl.BlockSpec((BSC, ROPE), lambda i: (i, 0)),
            pl.BlockSpec((BSC, NH * VDIM), lambda i: (i, 0)),
        ],
        out_shape=[
            jax.ShapeDtypeStruct((S, NH * NOPE), bf16),
            jax.ShapeDtypeStruct((S, ROPE), bf16),
            jax.ShapeDtypeStruct((S, NH * VDIM), bf16),
        ],
        scratch_shapes=[
            pltpu.VMEM((KVLORA + ROPE, DIM), bf16),
            pltpu.VMEM((NH * (NOPE + VDIM), KVLORA), bf16),
        ],
    )(x2, W_kva, W_kvb, gkv2, c64, s1_64, s2_64)

    o = pl.pallas_call(
        _attn_kernel,
        grid=(NH // 2,),
        in_specs=[
            pl.BlockSpec((S, 2 * QK), lambda h: (0, h)),
            pl.BlockSpec((S, 2 * NOPE), lambda h: (0, h)),
            pl.BlockSpec((S, ROPE), lambda h: (0, 0)),
            pl.BlockSpec((S, 2 * VDIM), lambda h: (0, h)),
        ],
        out_specs=pl.BlockSpec((S, 2 * VDIM), lambda h: (0, h)),
        out_shape=jax.ShapeDtypeStruct((S, NH * VDIM), bf16),
    )(q_int, kn, kp, v)

    out = pl.pallas_call(
        _oproj_kernel,
        grid=(S // BSO,),
        in_specs=[
            pl.BlockSpec((BSO, NH * VDIM), lambda i: (i, 0)),
            pl.BlockSpec((DIM, NH * VDIM), lambda i: (0, 0)),
        ],
        out_specs=pl.BlockSpec((BSO, DIM), lambda i: (i, 0)),
        out_shape=jax.ShapeDtypeStruct((S, DIM), f32),
        scratch_shapes=[pltpu.VMEM((DIM, NH * VDIM), bf16)],
    )(o, W_o)

    return out.reshape(b, s, DIM)


# speedup vs baseline: 1.3121x; 1.0551x over previous
import jax
import jax.numpy as jnp
from jax.experimental import pallas as pl
from jax.experimental.pallas import tpu as pltpu

DIM = 2048
NH = 16
QLORA = 1536
KVLORA = 512
NOPE = 128
ROPE = 64
VDIM = 128
QK = NOPE + ROPE
S = 2048
EPS = 1e-6
SCALE = QK ** (-0.5)

f32 = jnp.float32
bf16 = jnp.bfloat16

NT = (((1,), (1,)), ((), ()))

BSA = 1024
BSB = 512
BSC = 512
BQ = 512
BK = 512
BSO = 1024


def _rope(x, c, s1, s2):
    w = x.shape[-1]
    return (x * c + pltpu.roll(x, w - 1, 1) * s1 + pltpu.roll(x, 1, 1) * s2)


def _qa_kernel(x_ref, wqa_ref, gqa_ref, out_ref, wqa_s):
    @pl.when(pl.program_id(0) == 0)
    def _cast():
        wqa_s[...] = wqa_ref[...].astype(bf16)

    xb = x_ref[...].astype(bf16)
    qa = jax.lax.dot_general(xb, wqa_s[...], NT, preferred_element_type=f32)
    var = jnp.mean(qa * qa, axis=-1, keepdims=True)
    out_ref[...] = ((qa * jax.lax.rsqrt(var + EPS)) * gqa_ref[...]).astype(bf16)


QBN = 768


def _qb_kernel(qa_ref, wqb_ref, c_ref, s1_ref, s2_ref, out_ref, wqb_s):
    @pl.when(pl.program_id(0) == 0)
    def _cast():
        wqb_s[...] = wqb_ref[...].astype(bf16)

    q = jax.lax.dot_general(qa_ref[...], wqb_s[...], NT,
                            preferred_element_type=f32)
    c = jnp.tile(c_ref[...], (1, QBN // QK))
    s1 = jnp.tile(s1_ref[...], (1, QBN // QK))
    s2 = jnp.tile(s2_ref[...], (1, QBN // QK))
    for cix in range(NH * QK // QBN):
        out_ref[:, cix * QBN:(cix + 1) * QBN] = _rope(
            q[:, cix * QBN:(cix + 1) * QBN], c, s1, s2).astype(bf16)


def _kv_kernel(x_ref, wkva_ref, wkvb_ref, gkv_ref, tbl_ref,
               kn_ref, kp_ref, v_ref, wkva_s, wk_s, wv_s):
    @pl.when(pl.program_id(0) == 0)
    def _cast():
        wkva_s[...] = wkva_ref[...].astype(bf16)
        for h in range(NH):
            base = h * (NOPE + VDIM)
            wk_s[h * NOPE:(h + 1) * NOPE, :] = (
                wkvb_ref[base:base + NOPE, :].astype(bf16))
            wv_s[h * VDIM:(h + 1) * VDIM, :] = (
                wkvb_ref[base + NOPE:base + NOPE + VDIM, :].astype(bf16))

    xb = x_ref[...].astype(bf16)
    kva = jax.lax.dot_general(xb, wkva_s[...], NT, preferred_element_type=f32)
    lat = kva[:, :KVLORA]
    kpe = _rope(kva[:, KVLORA:], tbl_ref[:, :ROPE],
                tbl_ref[:, ROPE:2 * ROPE], tbl_ref[:, 2 * ROPE:])
    kp_ref[...] = kpe.astype(bf16)
    var = jnp.mean(lat * lat, axis=-1, keepdims=True)
    latb = ((lat * jax.lax.rsqrt(var + EPS)) * gkv_ref[...]).astype(bf16)
    kn_ref[...] = jax.lax.dot_general(
        latb, wk_s[...], NT, preferred_element_type=f32).astype(bf16)
    v_ref[...] = jax.lax.dot_general(
        latb, wv_s[...], NT, preferred_element_type=f32).astype(bf16)


def _attn_kernel(q_ref, kn_ref, kp_ref, v_ref, o_ref):
    kp = kp_ref[...]
    for hh in range(2):
        qh = q_ref[:, hh * QK:(hh + 1) * QK]
        kh = jnp.concatenate(
            [kn_ref[:, hh * NOPE:(hh + 1) * NOPE], kp], axis=1)
        vh = v_ref[:, hh * VDIM:(hh + 1) * VDIM]
        outs = []
        for i in range(S // BQ):
            q_i = qh[i * BQ:(i + 1) * BQ, :]
            svals = []
            for j in range(i + 1):
                k_j = kh[j * BK:(j + 1) * BK, :]
                sc = jax.lax.dot_general(q_i, k_j, NT,
                                         preferred_element_type=f32) * SCALE
                if j == i:
                    r = jax.lax.broadcasted_iota(jnp.int32, (BQ, BK), 0)
                    cidx = jax.lax.broadcasted_iota(jnp.int32, (BQ, BK), 1)
                    sc = jnp.where(cidx <= r, sc, -1e30)
                svals.append(sc)
            m = svals[0].max(axis=1, keepdims=True)
            for sv in svals[1:]:
                m = jnp.maximum(m, sv.max(axis=1, keepdims=True))
            ps = [jnp.exp(sv - m) for sv in svals]
            l = ps[0].sum(axis=1, keepdims=True)
            for p in ps[1:]:
                l = l + p.sum(axis=1, keepdims=True)
            acc = jnp.dot(ps[0].astype(bf16), vh[0:BK, :],
                          preferred_element_type=f32)
            for j in range(1, i + 1):
                acc = acc + jnp.dot(ps[j].astype(bf16),
                                    vh[j * BK:(j + 1) * BK, :],
                                    preferred_element_type=f32)
            outs.append((acc * (1.0 / l)).astype(bf16))
        o_ref[:, hh * VDIM:(hh + 1) * VDIM] = jnp.concatenate(outs, axis=0)


def _oproj_kernel(o_ref, wo_ref, out_ref, wo_s):
    @pl.when(pl.program_id(0) == 0)
    def _cast():
        wo_s[...] = wo_ref[...].astype(bf16)

    out_ref[...] = jax.lax.dot_general(o_ref[...], wo_s[...], NT,
                                       preferred_element_type=f32)


def kernel(x, freqs_cos, freqs_sin, mask, W_qa, g_qa, W_qb, W_kva, g_kv,
           W_kvb, W_o):
    del mask
    b, s, _ = x.shape
    x2 = x.reshape(s, DIM)
    gqa2 = g_qa.reshape(1, QLORA)
    gkv2 = g_kv.reshape(1, KVLORA)

    c64 = jnp.repeat(freqs_cos, 2, axis=-1)
    s64 = jnp.repeat(freqs_sin, 2, axis=-1)
    even = (jnp.arange(ROPE) % 2 == 0)
    s1_64 = jnp.where(even, -s64, 0.0)
    s2_64 = jnp.where(even, 0.0, s64)
    ones_n = jnp.ones((S, NOPE), f32)
    zero_n = jnp.zeros((S, NOPE), f32)
    c192 = jnp.concatenate([ones_n, c64], axis=1)
    s1_192 = jnp.concatenate([zero_n, s1_64], axis=1)
    s2_192 = jnp.concatenate([zero_n, s2_64], axis=1)
    tbl64 = jnp.concatenate([c64, s1_64, s2_64], axis=1)

    qa_n = pl.pallas_call(
        _qa_kernel,
        grid=(S // BSA,),
        in_specs=[
            pl.BlockSpec((BSA, DIM), lambda i: (i, 0)),
            pl.BlockSpec((QLORA, DIM), lambda i: (0, 0)),
            pl.BlockSpec((1, QLORA), lambda i: (0, 0)),
        ],
        out_specs=pl.BlockSpec((BSA, QLORA), lambda i: (i, 0)),
        out_shape=jax.ShapeDtypeStruct((S, QLORA), bf16),
        scratch_shapes=[pltpu.VMEM((QLORA, DIM), bf16)],
    )(x2, W_qa, gqa2)

    q_int = pl.pallas_call(
        _qb_kernel,
        grid=(S // BSB,),
        in_specs=[
            pl.BlockSpec((BSB, QLORA), lambda r: (r, 0)),
            pl.BlockSpec((NH * QK, QLORA), lambda r: (0, 0)),
            pl.BlockSpec((BSB, QK), lambda r: (r, 0)),
            pl.BlockSpec((BSB, QK), lambda r: (r, 0)),
            pl.BlockSpec((BSB, QK), lambda r: (r, 0)),
        ],
        out_specs=pl.BlockSpec((BSB, NH * QK), lambda r: (r, 0)),
        out_shape=jax.ShapeDtypeStruct((S, NH * QK), bf16),
        scratch_shapes=[pltpu.VMEM((NH * QK, QLORA), bf16)],
    )(qa_n, W_qb, c192, s1_192, s2_192)

    kn, kp, v = pl.pallas_call(
        _kv_kernel,
        grid=(S // BSC,),
        in_specs=[
            pl.BlockSpec((BSC, DIM), lambda i: (i, 0)),
            pl.BlockSpec((KVLORA + ROPE, DIM), lambda i: (0, 0)),
            pl.BlockSpec((NH * (NOPE + VDIM), KVLORA), lambda i: (0, 0)),
            pl.BlockSpec((1, KVLORA), lambda i: (0, 0)),
            pl.BlockSpec((BSC, 3 * ROPE), lambda i: (i, 0)),
        ],
        out_specs=[
            pl.BlockSpec((BSC, NH * NOPE), lambda i: (i, 0)),
            pl.BlockSpec((BSC, ROPE), lambda i: (i, 0)),
            pl.BlockSpec((BSC, NH * VDIM), lambda i: (i, 0)),
        ],
        out_shape=[
            jax.ShapeDtypeStruct((S, NH * NOPE), bf16),
            jax.ShapeDtypeStruct((S, ROPE), bf16),
            jax.ShapeDtypeStruct((S, NH * VDIM), bf16),
        ],
        scratch_shapes=[
            pltpu.VMEM((KVLORA + ROPE, DIM), bf16),
            pltpu.VMEM((NH * NOPE, KVLORA), bf16),
            pltpu.VMEM((NH * VDIM, KVLORA), bf16),
        ],
    )(x2, W_kva, W_kvb, gkv2, tbl64)

    o = pl.pallas_call(
        _attn_kernel,
        grid=(NH // 2,),
        in_specs=[
            pl.BlockSpec((S, 2 * QK), lambda h: (0, h)),
            pl.BlockSpec((S, 2 * NOPE), lambda h: (0, h)),
            pl.BlockSpec((S, ROPE), lambda h: (0, 0)),
            pl.BlockSpec((S, 2 * VDIM), lambda h: (0, h)),
        ],
        out_specs=pl.BlockSpec((S, 2 * VDIM), lambda h: (0, h)),
        out_shape=jax.ShapeDtypeStruct((S, NH * VDIM), bf16),
    )(q_int, kn, kp, v)

    out = pl.pallas_call(
        _oproj_kernel,
        grid=(S // BSO,),
        in_specs=[
            pl.BlockSpec((BSO, NH * VDIM), lambda i: (i, 0)),
            pl.BlockSpec((DIM, NH * VDIM), lambda i: (0, 0)),
        ],
        out_specs=pl.BlockSpec((BSO, DIM), lambda i: (i, 0)),
        out_shape=jax.ShapeDtypeStruct((S, DIM), f32),
        scratch_shapes=[pltpu.VMEM((DIM, NH * VDIM), bf16)],
    )(o, W_o)

    return out.reshape(b, s, DIM)


# speedup vs baseline: 1.3467x; 1.0264x over previous
import jax
import jax.numpy as jnp
from jax.experimental import pallas as pl
from jax.experimental.pallas import tpu as pltpu

DIM = 2048
NH = 16
QLORA = 1536
KVLORA = 512
NOPE = 128
ROPE = 64
VDIM = 128
QK = NOPE + ROPE
S = 2048
EPS = 1e-6
SCALE = QK ** (-0.5)

f32 = jnp.float32
bf16 = jnp.bfloat16

NT = (((1,), (1,)), ((), ()))

BSA = 1024
BSB = 512
BSC = 512
BQ = 512
BK = 512
BSO = 1024


def _rope(x, c, s1, s2):
    w = x.shape[-1]
    return (x * c + pltpu.roll(x, w - 1, 1) * s1 + pltpu.roll(x, 1, 1) * s2)


def _qa_kernel(x_ref, wqa_ref, gqa_ref, out_ref, wqa_s):
    @pl.when(pl.program_id(0) == 0)
    def _cast():
        wqa_s[...] = wqa_ref[...].astype(bf16)

    xb = x_ref[...].astype(bf16)
    qa = jax.lax.dot_general(xb, wqa_s[...], NT, preferred_element_type=f32)
    var = jnp.mean(qa * qa, axis=-1, keepdims=True)
    out_ref[...] = ((qa * jax.lax.rsqrt(var + EPS)) * gqa_ref[...]).astype(bf16)


QBN = 768


def _qb_kernel(qa_ref, wqb_ref, c_ref, s1_ref, s2_ref, out_ref, wqb_s):
    @pl.when(pl.program_id(0) == 0)
    def _cast():
        wqb_s[...] = wqb_ref[...].astype(bf16)

    q = jax.lax.dot_general(qa_ref[...], wqb_s[...], NT,
                            preferred_element_type=f32)
    c = jnp.tile(c_ref[...], (1, QBN // QK))
    s1 = jnp.tile(s1_ref[...], (1, QBN // QK))
    s2 = jnp.tile(s2_ref[...], (1, QBN // QK))
    for cix in range(NH * QK // QBN):
        out_ref[:, cix * QBN:(cix + 1) * QBN] = (
            _rope(q[:, cix * QBN:(cix + 1) * QBN], c, s1, s2)
            * (SCALE * 1.4426950408889634)).astype(bf16)


def _kv_kernel(x_ref, wkva_ref, wkvb_ref, gkv_ref, tbl_ref,
               kn_ref, kp_ref, v_ref, wkva_s, wk_s, wv_s):
    @pl.when(pl.program_id(0) == 0)
    def _cast():
        wkva_s[...] = wkva_ref[...].astype(bf16)
        for h in range(NH):
            base = h * (NOPE + VDIM)
            wk_s[h * NOPE:(h + 1) * NOPE, :] = (
                wkvb_ref[base:base + NOPE, :].astype(bf16))
            wv_s[h * VDIM:(h + 1) * VDIM, :] = (
                wkvb_ref[base + NOPE:base + NOPE + VDIM, :].astype(bf16))

    xb = x_ref[...].astype(bf16)
    kva = jax.lax.dot_general(xb, wkva_s[...], NT, preferred_element_type=f32)
    lat = kva[:, :KVLORA]
    kpe = _rope(kva[:, KVLORA:], tbl_ref[:, :ROPE],
                tbl_ref[:, ROPE:2 * ROPE], tbl_ref[:, 2 * ROPE:])
    kp_ref[...] = kpe.astype(bf16)
    var = jnp.mean(lat * lat, axis=-1, keepdims=True)
    latb = ((lat * jax.lax.rsqrt(var + EPS)) * gkv_ref[...]).astype(bf16)
    kn_ref[...] = jax.lax.dot_general(
        latb, wk_s[...], NT, preferred_element_type=f32).astype(bf16)
    v_ref[...] = jax.lax.dot_general(
        latb, wv_s[...], NT, preferred_element_type=f32).astype(bf16)


def _attn_kernel(q_ref, kn_ref, kp_ref, v_ref, o_ref):
    kp = kp_ref[...]
    for hh in range(2):
        qh = q_ref[:, hh * QK:(hh + 1) * QK]
        kh = jnp.concatenate(
            [kn_ref[:, hh * NOPE:(hh + 1) * NOPE], kp], axis=1)
        vh = v_ref[:, hh * VDIM:(hh + 1) * VDIM]
        outs = []
        for i in range(S // BQ):
            q_i = qh[i * BQ:(i + 1) * BQ, :]
            svals = []
            for j in range(i + 1):
                k_j = kh[j * BK:(j + 1) * BK, :]
                sc = jax.lax.dot_general(q_i, k_j, NT,
                                         preferred_element_type=f32)
                if j == i:
                    r = jax.lax.broadcasted_iota(jnp.int32, (BQ, BK), 0)
                    cidx = jax.lax.broadcasted_iota(jnp.int32, (BQ, BK), 1)
                    sc = jnp.where(cidx <= r, sc, -1e30)
                svals.append(sc)
            m = svals[0].max(axis=1, keepdims=True)
            for sv in svals[1:]:
                m = jnp.maximum(m, sv.max(axis=1, keepdims=True))
            ps = [jnp.exp2(sv - m) for sv in svals]
            l = ps[0].sum(axis=1, keepdims=True)
            for p in ps[1:]:
                l = l + p.sum(axis=1, keepdims=True)
            acc = jnp.dot(ps[0].astype(bf16), vh[0:BK, :],
                          preferred_element_type=f32)
            for j in range(1, i + 1):
                acc = acc + jnp.dot(ps[j].astype(bf16),
                                    vh[j * BK:(j + 1) * BK, :],
                                    preferred_element_type=f32)
            outs.append((acc * (1.0 / l)).astype(bf16))
        o_ref[:, hh * VDIM:(hh + 1) * VDIM] = jnp.concatenate(outs, axis=0)


def _oproj_kernel(o_ref, wo_ref, out_ref, wo_s):
    @pl.when(pl.program_id(0) == 0)
    def _cast():
        wo_s[...] = wo_ref[...].astype(bf16)

    out_ref[...] = jax.lax.dot_general(o_ref[...], wo_s[...], NT,
                                       preferred_element_type=f32)


def kernel(x, freqs_cos, freqs_sin, mask, W_qa, g_qa, W_qb, W_kva, g_kv,
           W_kvb, W_o):
    del mask
    b, s, _ = x.shape
    x2 = x.reshape(s, DIM)
    gqa2 = g_qa.reshape(1, QLORA)
    gkv2 = g_kv.reshape(1, KVLORA)

    c64 = jnp.repeat(freqs_cos, 2, axis=-1)
    s64 = jnp.repeat(freqs_sin, 2, axis=-1)
    even = (jnp.arange(ROPE) % 2 == 0)
    s1_64 = jnp.where(even, -s64, 0.0)
    s2_64 = jnp.where(even, 0.0, s64)
    ones_n = jnp.ones((S, NOPE), f32)
    zero_n = jnp.zeros((S, NOPE), f32)
    c192 = jnp.concatenate([ones_n, c64], axis=1)
    s1_192 = jnp.concatenate([zero_n, s1_64], axis=1)
    s2_192 = jnp.concatenate([zero_n, s2_64], axis=1)
    tbl64 = jnp.concatenate([c64, s1_64, s2_64], axis=1)

    qa_n = pl.pallas_call(
        _qa_kernel,
        grid=(S // BSA,),
        in_specs=[
            pl.BlockSpec((BSA, DIM), lambda i: (i, 0)),
            pl.BlockSpec((QLORA, DIM), lambda i: (0, 0)),
            pl.BlockSpec((1, QLORA), lambda i: (0, 0)),
        ],
        out_specs=pl.BlockSpec((BSA, QLORA), lambda i: (i, 0)),
        out_shape=jax.ShapeDtypeStruct((S, QLORA), bf16),
        scratch_shapes=[pltpu.VMEM((QLORA, DIM), bf16)],
    )(x2, W_qa, gqa2)

    q_int = pl.pallas_call(
        _qb_kernel,
        grid=(S // BSB,),
        in_specs=[
            pl.BlockSpec((BSB, QLORA), lambda r: (r, 0)),
            pl.BlockSpec((NH * QK, QLORA), lambda r: (0, 0)),
            pl.BlockSpec((BSB, QK), lambda r: (r, 0)),
            pl.BlockSpec((BSB, QK), lambda r: (r, 0)),
            pl.BlockSpec((BSB, QK), lambda r: (r, 0)),
        ],
        out_specs=pl.BlockSpec((BSB, NH * QK), lambda r: (r, 0)),
        out_shape=jax.ShapeDtypeStruct((S, NH * QK), bf16),
        scratch_shapes=[pltpu.VMEM((NH * QK, QLORA), bf16)],
    )(qa_n, W_qb, c192, s1_192, s2_192)

    kn, kp, v = pl.pallas_call(
        _kv_kernel,
        grid=(S // BSC,),
        in_specs=[
            pl.BlockSpec((BSC, DIM), lambda i: (i, 0)),
            pl.BlockSpec((KVLORA + ROPE, DIM), lambda i: (0, 0)),
            pl.BlockSpec((NH * (NOPE + VDIM), KVLORA), lambda i: (0, 0)),
            pl.BlockSpec((1, KVLORA), lambda i: (0, 0)),
            pl.BlockSpec((BSC, 3 * ROPE), lambda i: (i, 0)),
        ],
        out_specs=[
            pl.BlockSpec((BSC, NH * NOPE), lambda i: (i, 0)),
            pl.BlockSpec((BSC, ROPE), lambda i: (i, 0)),
            pl.BlockSpec((BSC, NH * VDIM), lambda i: (i, 0)),
        ],
        out_shape=[
            jax.ShapeDtypeStruct((S, NH * NOPE), bf16),
            jax.ShapeDtypeStruct((S, ROPE), bf16),
            jax.ShapeDtypeStruct((S, NH * VDIM), bf16),
        ],
        scratch_shapes=[
            pltpu.VMEM((KVLORA + ROPE, DIM), bf16),
            pltpu.VMEM((NH * NOPE, KVLORA), bf16),
            pltpu.VMEM((NH * VDIM, KVLORA), bf16),
        ],
    )(x2, W_kva, W_kvb, gkv2, tbl64)

    o = pl.pallas_call(
        _attn_kernel,
        grid=(NH // 2,),
        in_specs=[
            pl.BlockSpec((S, 2 * QK), lambda h: (0, h)),
            pl.BlockSpec((S, 2 * NOPE), lambda h: (0, h)),
            pl.BlockSpec((S, ROPE), lambda h: (0, 0)),
            pl.BlockSpec((S, 2 * VDIM), lambda h: (0, h)),
        ],
        out_specs=pl.BlockSpec((S, 2 * VDIM), lambda h: (0, h)),
        out_shape=jax.ShapeDtypeStruct((S, NH * VDIM), bf16),
    )(q_int, kn, kp, v)

    out = pl.pallas_call(
        _oproj_kernel,
        grid=(S // BSO,),
        in_specs=[
            pl.BlockSpec((BSO, NH * VDIM), lambda i: (i, 0)),
            pl.BlockSpec((DIM, NH * VDIM), lambda i: (0, 0)),
        ],
        out_specs=pl.BlockSpec((BSO, DIM), lambda i: (i, 0)),
        out_shape=jax.ShapeDtypeStruct((S, DIM), f32),
        scratch_shapes=[pltpu.VMEM((DIM, NH * VDIM), bf16)],
    )(o, W_o)

    return out.reshape(b, s, DIM)
